# baseline (speedup 1.0000x reference)
import functools

import jax
import jax.numpy as jnp
from jax import lax
from jax.experimental import pallas as pl
from jax.experimental.pallas import tpu as pltpu
from jax.experimental.pallas import tpu_sc as plsc

_DEPTH_SCALES = 4


_NBUF = 8


def _sc_gather_call(table1d, idxg, n_rows, row_words):
    info = plsc.get_sparse_core_info()
    nw = info.num_cores * info.num_subcores
    bpw = n_rows // nw
    m_groups = bpw // _NBUF
    mesh = plsc.VectorSubcoreMesh(core_axis_name="c", subcore_axis_name="s")

    @functools.partial(
        pl.kernel,
        out_type=jax.ShapeDtypeStruct((n_rows * row_words,), jnp.float32),
        mesh=mesh,
        scratch_types=[
            pltpu.SMEM((bpw,), jnp.int32),
            pltpu.VMEM_SHARED((16, bpw), jnp.int32),
            *[pltpu.VMEM((row_words,), jnp.float32) for _ in range(_NBUF)],
            pltpu.SemaphoreType.DMA((_NBUF,)),
            pltpu.SemaphoreType.DMA((_NBUF,)),
        ],
    )
    def k(table_hbm, idx_hbm, out_hbm, idx_s, idx_v, *rest):
        bufs = rest[:_NBUF]
        gsem, ssem = rest[_NBUF], rest[_NBUF + 1]
        sid = lax.axis_index("s")
        wid = sid * info.num_cores + lax.axis_index("c")
        base = wid * bpw
        pltpu.sync_copy(idx_hbm.at[wid], idx_v.at[sid])
        pltpu.sync_copy(idx_v.at[sid], idx_s)

        def gather(row, b):
            src = table_hbm.at[pl.ds(idx_s[row] * row_words, row_words)]
            return pltpu.make_async_copy(src, bufs[b], gsem.at[b])

        def scatter(row, b):
            dst = out_hbm.at[pl.ds((base + row) * row_words, row_words)]
            return pltpu.make_async_copy(bufs[b], dst, ssem.at[b])

        for b in range(_NBUF):
            gather(b, b).start()

        @pl.loop(0, m_groups)
        def _(m):
            for b in range(_NBUF):
                row = m * _NBUF + b
                gather(row, b).wait()
                scatter(row, b).start()
            for b in range(_NBUF):
                row = m * _NBUF + b
                scatter(row, b).wait()

                @pl.when(m < m_groups - 1)
                def _():
                    gather(row + _NBUF, b).start()

    return k(table1d, idxg.reshape(nw, bpw))


def kernel(x, W1, b1, W2, b2):
    n, c, h, w = x.shape
    d = _DEPTH_SCALES
    block_size = c // d
    avg = jnp.mean(x, axis=(2, 3))
    mx = jnp.max(x, axis=(2, 3))

    def mlp(v):
        hdn = jnp.maximum(v @ W1 + b1, 0.0)
        return hdn @ W2 + b2

    attn = jax.nn.sigmoid(mlp(avg) + mlp(mx))
    attn = attn.reshape(n, c, d)
    attn_t = jnp.transpose(attn, (0, 2, 1))
    _, idx = jax.lax.top_k(attn_t, block_size)
    idx_flat = idx.reshape(n, d * block_size).astype(jnp.int32)

    idxg = (jnp.arange(n, dtype=jnp.int32)[:, None] * c + idx_flat).reshape(-1)
    out = _sc_gather_call(x.reshape(-1), idxg, n * c, h * w)
    return out.reshape(n, c, h, w)


# speedup vs baseline: 1.0066x; 1.0066x over previous
import functools

import jax
import jax.numpy as jnp
from jax import lax
from jax.experimental import pallas as pl
from jax.experimental.pallas import tpu as pltpu
from jax.experimental.pallas import tpu_sc as plsc

_DEPTH_SCALES = 4

_CHUNK = 8


def _sc_gather_call(table1d, idxg, n_rows, row_words):
    info = plsc.get_sparse_core_info()
    nw = info.num_cores * info.num_subcores
    bpw = n_rows // nw
    n_chunks = bpw // _CHUNK
    n_pairs = n_chunks // 2
    mesh = plsc.VectorSubcoreMesh(core_axis_name="c", subcore_axis_name="s")

    @functools.partial(
        pl.kernel,
        out_type=jax.ShapeDtypeStruct((n_rows * row_words,), jnp.float32),
        mesh=mesh,
        scratch_types=[
            pltpu.SMEM((bpw,), jnp.int32),
            pltpu.VMEM_SHARED((16, bpw), jnp.int32),
            pltpu.VMEM((_CHUNK * row_words,), jnp.float32),
            pltpu.VMEM((_CHUNK * row_words,), jnp.float32),
            pltpu.SemaphoreType.DMA((2, _CHUNK)),
            pltpu.SemaphoreType.DMA((2,)),
        ],
    )
    def k(table_hbm, idx_hbm, out_hbm, idx_s, idx_v, buf0, buf1, gsem, ssem):
        sid = lax.axis_index("s")
        wid = sid * info.num_cores + lax.axis_index("c")
        base = wid * bpw
        pltpu.sync_copy(idx_hbm.at[wid], idx_v.at[sid])
        pltpu.sync_copy(idx_v.at[sid], idx_s)
        bufs = (buf0, buf1)

        def gather(ch, p, j):
            row = ch * _CHUNK + j
            src = table_hbm.at[pl.ds(idx_s[row] * row_words, row_words)]
            dst = bufs[p].at[pl.ds(j * row_words, row_words)]
            return pltpu.make_async_copy(src, dst, gsem.at[p, j])

        def scatter(ch, p):
            dst = out_hbm.at[pl.ds((base + ch * _CHUNK) * row_words,
                                   _CHUNK * row_words)]
            return pltpu.make_async_copy(bufs[p], dst, ssem.at[p])

        for j in range(_CHUNK):
            gather(0, 0, j).start()

        @pl.loop(0, n_pairs)
        def _(m):
            ch0 = 2 * m
            ch1 = 2 * m + 1

            @pl.when(m > 0)
            def _():
                scatter(ch1 - 2, 1).wait()

            for j in range(_CHUNK):
                gather(ch1, 1, j).start()

            for j in range(_CHUNK):
                gather(ch0, 0, j).wait()
            scatter(ch0, 0).start()

            @pl.when(m < n_pairs - 1)
            def _():
                scatter(ch0, 0).wait()
                for j in range(_CHUNK):
                    gather(ch0 + 2, 0, j).start()

            for j in range(_CHUNK):
                gather(ch1, 1, j).wait()
            scatter(ch1, 1).start()

        scatter(n_chunks - 2, 0).wait()
        scatter(n_chunks - 1, 1).wait()

    return k(table1d, idxg.reshape(nw, bpw))


def kernel(x, W1, b1, W2, b2):
    n, c, h, w = x.shape
    d = _DEPTH_SCALES
    block_size = c // d
    avg = jnp.mean(x, axis=(2, 3))
    mx = jnp.max(x, axis=(2, 3))

    def mlp(v):
        hdn = jnp.maximum(v @ W1 + b1, 0.0)
        return hdn @ W2 + b2

    attn = jax.nn.sigmoid(mlp(avg) + mlp(mx))
    attn = attn.reshape(n, c, d)
    attn_t = jnp.transpose(attn, (0, 2, 1))
    _, idx = jax.lax.top_k(attn_t, block_size)
    idx_flat = idx.reshape(n, d * block_size).astype(jnp.int32)

    idxg = (jnp.arange(n, dtype=jnp.int32)[:, None] * c + idx_flat).reshape(-1)
    out = _sc_gather_call(x.reshape(-1), idxg, n * c, h * w)
    return out.reshape(n, c, h, w)
